# baseline (speedup 1.0000x reference)
import functools

import jax
import jax.numpy as jnp
from jax import lax
from jax.experimental import pallas as pl
from jax.experimental.pallas import tpu as pltpu
from jax.experimental.pallas import tpu_sc as plsc

N_CODES = 8192
DIM = 256
B = 32768


M_BLK = 1024
M_STEPS = B // M_BLK


LG = 128
NG = N_CODES // LG


def _argmin_body(z_ref, et_ref, en_ref, idx_ref, loss_ref):
    z = z_ref[...]
    zn = jnp.sum(z * z, axis=1, keepdims=True)
    zm2 = (z * (-2.0)).astype(jnp.bfloat16)
    en = en_ref[...]
    dots = lax.dot_general(zm2, et_ref[...], (((1,), (0,)), ((), ())),
                           preferred_element_type=jnp.float32)
    run_min = jnp.full((M_BLK, LG), jnp.inf, jnp.float32)
    run_grp = jnp.zeros((M_BLK, LG), jnp.int32)
    for g in range(NG):
        d = (zn + dots[:, g * LG:(g + 1) * LG]) + en[:, g * LG:(g + 1) * LG]
        upd = d < run_min
        run_min = jnp.where(upd, d, run_min)
        run_grp = jnp.where(upd, g, run_grp)
    minv = jnp.min(run_min, axis=1, keepdims=True)
    lane = lax.broadcasted_iota(jnp.int32, (M_BLK, LG), 1)
    cand = jnp.where(run_min == minv, run_grp * LG + lane, N_CODES)
    idx_ref[0, 0, :] = jnp.min(cand, axis=1)
    loss_ref[0, 0, 0] = jnp.sum(minv) * (1.0 / (B * DIM))


def _argmin_call(z_e, et_bf, en):
    return pl.pallas_call(
        _argmin_body,
        grid=(M_STEPS,),
        in_specs=[
            pl.BlockSpec((M_BLK, DIM), lambda m: (m, 0)),
            pl.BlockSpec((DIM, N_CODES), lambda m: (0, 0)),
            pl.BlockSpec((1, N_CODES), lambda m: (0, 0)),
        ],
        out_specs=[
            pl.BlockSpec((1, 1, M_BLK), lambda m: (m, 0, 0)),
            pl.BlockSpec(memory_space=pltpu.SMEM, block_shape=(1, 1, 1),
                         index_map=lambda m: (m, 0, 0)),
        ],
        out_shape=[
            jax.ShapeDtypeStruct((M_STEPS, 1, M_BLK), jnp.int32),
            jax.ShapeDtypeStruct((M_STEPS, 1, 1), jnp.float32),
        ],
        compiler_params=pltpu.CompilerParams(
            dimension_semantics=("parallel",)),
    )(z_e, et_bf, en)


NW = 32
BPW = B // NW
CH = 64
NCH = BPW // CH


NB = 4


@functools.cache
def _gather_sc():
    @functools.partial(
        pl.kernel,
        mesh=plsc.VectorSubcoreMesh(core_axis_name="c", subcore_axis_name="s"),
        out_type=jax.ShapeDtypeStruct((B, DIM), jnp.float32),
        scratch_types=[
            pltpu.VMEM((NCH, CH), jnp.int32),
            [pltpu.VMEM((CH, DIM), jnp.float32) for _ in range(NB)],
            [pltpu.SemaphoreType.DMA for _ in range(NB)],
            [pltpu.SemaphoreType.DMA for _ in range(NB)],
        ],
    )
    def gather(table_hbm, idx_hbm, out_hbm, idx_v, rows, gsems, wsems):
        wid = lax.axis_index("s") * 2 + lax.axis_index("c")
        base = wid * BPW
        pltpu.sync_copy(idx_hbm.at[wid], idx_v)
        ghandles = [None] * NCH
        whandles = [None] * NCH
        for j in range(NCH):
            b = j % NB
            if j >= NB:
                whandles[j - NB].wait()
            ghandles[j] = pltpu.async_copy(
                table_hbm.at[idx_v.at[j]], rows[b], gsems[b])
            if j >= 1:
                bp = (j - 1) % NB
                ghandles[j - 1].wait()
                whandles[j - 1] = pltpu.async_copy(
                    rows[bp], out_hbm.at[pl.ds(base + (j - 1) * CH, CH)],
                    wsems[bp])
        ghandles[NCH - 1].wait()
        whandles[NCH - 1] = pltpu.async_copy(
            rows[(NCH - 1) % NB],
            out_hbm.at[pl.ds(base + (NCH - 1) * CH, CH)],
            wsems[(NCH - 1) % NB])
        for j in range(NCH - NB, NCH):
            whandles[j].wait()

    return gather


def kernel(z_e, embeddings):
    en = jnp.sum(embeddings ** 2, axis=1).reshape(1, N_CODES)
    idx3d, loss = _argmin_call(z_e, embeddings.T.astype(jnp.bfloat16), en)
    indices = idx3d.reshape(B)
    z_q = _gather_sc()(embeddings, indices.reshape(NW, NCH, CH))
    return z_q, indices, jnp.sum(loss.reshape(M_STEPS))


# speedup vs baseline: 1.0219x; 1.0219x over previous
import functools

import jax
import jax.numpy as jnp
from jax import lax
from jax.experimental import pallas as pl
from jax.experimental.pallas import tpu as pltpu
from jax.experimental.pallas import tpu_sc as plsc

N_CODES = 8192
DIM = 256
B = 32768


M_BLK = 1024
M_STEPS = B // M_BLK


LG = 128
NG = N_CODES // LG


def _prep_body(e_ref, et_ref, en_ref):
    et32 = e_ref[...].T
    et_ref[...] = et32.astype(jnp.bfloat16)
    en_ref[...] = jnp.sum(et32 * et32, axis=0, keepdims=True)


def _prep_call(embeddings):
    return pl.pallas_call(
        _prep_body,
        out_shape=[
            jax.ShapeDtypeStruct((DIM, N_CODES), jnp.bfloat16),
            jax.ShapeDtypeStruct((1, N_CODES), jnp.float32),
        ],
    )(embeddings)


def _argmin_body(z_ref, et_ref, en_ref, idx_ref, loss_ref):
    z = z_ref[...]
    zn = jnp.sum(z * z, axis=1, keepdims=True)
    zm2 = (z * (-2.0)).astype(jnp.bfloat16)
    en = en_ref[...]
    dots = lax.dot_general(zm2, et_ref[...], (((1,), (0,)), ((), ())),
                           preferred_element_type=jnp.float32)
    run_min = jnp.full((M_BLK, LG), jnp.inf, jnp.float32)
    run_grp = jnp.zeros((M_BLK, LG), jnp.int32)
    for g in range(NG):
        d = (zn + dots[:, g * LG:(g + 1) * LG]) + en[:, g * LG:(g + 1) * LG]
        upd = d < run_min
        run_min = jnp.where(upd, d, run_min)
        run_grp = jnp.where(upd, g, run_grp)
    minv = jnp.min(run_min, axis=1, keepdims=True)
    lane = lax.broadcasted_iota(jnp.int32, (M_BLK, LG), 1)
    cand = jnp.where(run_min == minv, run_grp * LG + lane, N_CODES)
    idx_ref[0, 0, :] = jnp.min(cand, axis=1)
    loss_ref[0, 0, 0] = jnp.sum(minv) * (1.0 / (B * DIM))


def _argmin_call(z_e, et_bf, en):
    return pl.pallas_call(
        _argmin_body,
        grid=(M_STEPS,),
        in_specs=[
            pl.BlockSpec((M_BLK, DIM), lambda m: (m, 0)),
            pl.BlockSpec((DIM, N_CODES), lambda m: (0, 0)),
            pl.BlockSpec((1, N_CODES), lambda m: (0, 0)),
        ],
        out_specs=[
            pl.BlockSpec((1, 1, M_BLK), lambda m: (m, 0, 0)),
            pl.BlockSpec(memory_space=pltpu.SMEM, block_shape=(1, 1, 1),
                         index_map=lambda m: (m, 0, 0)),
        ],
        out_shape=[
            jax.ShapeDtypeStruct((M_STEPS, 1, M_BLK), jnp.int32),
            jax.ShapeDtypeStruct((M_STEPS, 1, 1), jnp.float32),
        ],
        compiler_params=pltpu.CompilerParams(
            dimension_semantics=("parallel",)),
    )(z_e, et_bf, en)


NW = 32
BPW = B // NW
CH = 128
NCH = BPW // CH


NB = 3


@functools.cache
def _gather_sc():
    @functools.partial(
        pl.kernel,
        mesh=plsc.VectorSubcoreMesh(core_axis_name="c", subcore_axis_name="s"),
        out_type=jax.ShapeDtypeStruct((B, DIM), jnp.float32),
        scratch_types=[
            pltpu.VMEM((NCH, CH), jnp.int32),
            [pltpu.VMEM((CH, DIM), jnp.float32) for _ in range(NB)],
            [pltpu.SemaphoreType.DMA for _ in range(NB)],
            [pltpu.SemaphoreType.DMA for _ in range(NB)],
        ],
    )
    def gather(table_hbm, idx_hbm, out_hbm, idx_v, rows, gsems, wsems):
        wid = lax.axis_index("s") * 2 + lax.axis_index("c")
        base = wid * BPW
        pltpu.sync_copy(idx_hbm.at[wid], idx_v)
        ghandles = [None] * NCH
        whandles = [None] * NCH
        for j in range(NCH):
            b = j % NB
            if j >= NB:
                whandles[j - NB].wait()
            ghandles[j] = pltpu.async_copy(
                table_hbm.at[idx_v.at[j]], rows[b], gsems[b])
            if j >= 1:
                bp = (j - 1) % NB
                ghandles[j - 1].wait()
                whandles[j - 1] = pltpu.async_copy(
                    rows[bp], out_hbm.at[pl.ds(base + (j - 1) * CH, CH)],
                    wsems[bp])
        ghandles[NCH - 1].wait()
        whandles[NCH - 1] = pltpu.async_copy(
            rows[(NCH - 1) % NB],
            out_hbm.at[pl.ds(base + (NCH - 1) * CH, CH)],
            wsems[(NCH - 1) % NB])
        for j in range(NCH - NB, NCH):
            whandles[j].wait()

    return gather


def kernel(z_e, embeddings):
    et_bf, en = _prep_call(embeddings)
    idx3d, loss = _argmin_call(z_e, et_bf, en)
    indices = idx3d.reshape(B)
    z_q = _gather_sc()(embeddings, indices.reshape(NW, NCH, CH))
    return z_q, indices, jnp.sum(loss.reshape(M_STEPS))
